# Initial kernel scaffold; baseline (speedup 1.0000x reference)
import functools

import jax
import jax.numpy as jnp
from jax import lax
from jax.experimental import pallas as pl
from jax.experimental.pallas import tpu as pltpu
from jax.experimental.pallas import tpu_sc as plsc

NC = 2
NS = 16
CHUNK = 128

_mesh = functools.partial(
    plsc.VectorSubcoreMesh, core_axis_name="c", subcore_axis_name="s")


def _round_up(x, m):
  return (x + m - 1) // m * m


def _make_emb_gather(NPAD, H, GCH):
  nch = NPAD // (NC * NS * GCH)

  @functools.partial(
      pl.kernel,
      out_type=jax.ShapeDtypeStruct((NPAD, H), jnp.float32),
      mesh=_mesh(),
      scratch_types=[
          pltpu.VMEM((GCH,), jnp.int32),
          pltpu.VMEM((GCH, H), jnp.float32),
          pltpu.SemaphoreType.DMA,
      ],
  )
  def emb_gather(table, idx, out, idx_v, rows_v, sem):
    wid = lax.axis_index("s") * NC + lax.axis_index("c")

    def body(i, carry):
      base = (wid * nch + i) * GCH
      pltpu.sync_copy(idx.at[pl.ds(base, GCH)], idx_v)
      pltpu.async_copy(table.at[idx_v], rows_v, sem).wait()
      pltpu.sync_copy(rows_v, out.at[pl.ds(base, GCH)])
      return carry

    lax.fori_loop(0, nch, body, 0)

  return emb_gather


def _make_seg_sum(NPAD, H, EP, with_counts):
  nch = EP // (NS * CHUNK)
  rpt = NPAD // NS

  outs = [jax.ShapeDtypeStruct((NPAD, H), jnp.float32),
          jax.ShapeDtypeStruct((NPAD, H), jnp.float32)]
  scratch = [
      pltpu.VMEM_SHARED((NPAD, H), jnp.float32),
      pltpu.VMEM((CHUNK,), jnp.int32),
      pltpu.VMEM((CHUNK,), jnp.int32),
      pltpu.VMEM((CHUNK, H), jnp.float32),
      pltpu.SemaphoreType.DMA,
  ]
  if with_counts:
    outs += [jax.ShapeDtypeStruct((NPAD, 16), jnp.float32),
             jax.ShapeDtypeStruct((NPAD, 16), jnp.float32)]
    scratch += [pltpu.VMEM_SHARED((NPAD, 16), jnp.float32),
                pltpu.VMEM((CHUNK, 16), jnp.float32)]

  def body(feat_a, feat_b, srcp, dstp, zrows, zcnt, ones, *rest):
    if with_counts:
      (out_a, out_b, cnt_a, cnt_b,
       accum, gidx, sidx, rows, sem, cnt_sh, ones_v) = rest
    else:
      out_a, out_b, accum, gidx, sidx, rows, sem = rest
    c = lax.axis_index("c")
    sid = lax.axis_index("s")
    r0 = sid * rpt

    pltpu.sync_copy(zrows.at[pl.ds(r0, rpt)], accum.at[pl.ds(r0, rpt)])
    if with_counts:
      pltpu.sync_copy(zcnt.at[pl.ds(r0, rpt)], cnt_sh.at[pl.ds(r0, rpt)])
      pltpu.sync_copy(ones, ones_v)
    plsc.subcore_barrier()

    def cbody(ch, carry):
      base = (sid * nch + ch) * CHUNK

      @pl.when(c == 0)
      def _():
        pltpu.sync_copy(srcp.at[pl.ds(base, CHUNK)], gidx)
        pltpu.sync_copy(dstp.at[pl.ds(base, CHUNK)], sidx)
        pltpu.async_copy(feat_a.at[gidx], rows, sem).wait()

      @pl.when(c == 1)
      def _():
        pltpu.sync_copy(dstp.at[pl.ds(base, CHUNK)], gidx)
        pltpu.sync_copy(srcp.at[pl.ds(base, CHUNK)], sidx)
        pltpu.async_copy(feat_b.at[gidx], rows, sem).wait()

      pltpu.sync_copy(rows, accum.at[sidx], add=True)
      if with_counts:
        pltpu.sync_copy(ones_v, cnt_sh.at[sidx], add=True)
      return carry

    lax.fori_loop(0, nch, cbody, 0)
    plsc.subcore_barrier()

    @pl.when(c == 0)
    def _():
      pltpu.sync_copy(accum.at[pl.ds(r0, rpt)], out_a.at[pl.ds(r0, rpt)])
      if with_counts:
        pltpu.sync_copy(cnt_sh.at[pl.ds(r0, rpt)], cnt_a.at[pl.ds(r0, rpt)])

    @pl.when(c == 1)
    def _():
      pltpu.sync_copy(accum.at[pl.ds(r0, rpt)], out_b.at[pl.ds(r0, rpt)])
      if with_counts:
        pltpu.sync_copy(cnt_sh.at[pl.ds(r0, rpt)], cnt_b.at[pl.ds(r0, rpt)])

  return pl.kernel(body, out_type=outs, mesh=_mesh(), scratch_types=scratch)


def _make_classifier(NPAD, H, EP):
  nch = EP // (NC * NS * CHUNK)
  ngrp = CHUNK // 16

  @functools.partial(
      pl.kernel,
      out_type=jax.ShapeDtypeStruct((EP,), jnp.float32),
      mesh=_mesh(),
      scratch_types=[
          pltpu.VMEM((CHUNK,), jnp.int32),
          pltpu.VMEM((CHUNK,), jnp.int32),
          pltpu.VMEM((CHUNK, H), jnp.float32),
          pltpu.VMEM((CHUNK, H), jnp.float32),
          pltpu.VMEM((CHUNK,), jnp.float32),
          pltpu.SemaphoreType.DMA,
          pltpu.SemaphoreType.DMA,
      ],
  )
  def classifier(xd, xa, el0, el1, pred, i0v, i1v, rd, ra, pv, s0, s1):
    wid = lax.axis_index("s") * NC + lax.axis_index("c")

    def cbody(ch, carry):
      base = (wid * nch + ch) * CHUNK
      pltpu.sync_copy(el0.at[pl.ds(base, CHUNK)], i0v)
      pltpu.sync_copy(el1.at[pl.ds(base, CHUNK)], i1v)
      cp0 = pltpu.async_copy(xd.at[i0v], rd, s0)
      cp1 = pltpu.async_copy(xa.at[i1v], ra, s1)
      cp0.wait()
      cp1.wait()

      def gbody(g, carry2):
        r16 = g * 16 + lax.iota(jnp.int32, 16)
        acc = jnp.zeros((16,), jnp.float32)
        for f in range(H):
          fv = jnp.full((16,), f, jnp.int32)
          acc = acc + (plsc.load_gather(rd, [r16, fv]) *
                       plsc.load_gather(ra, [r16, fv]))
        pv[pl.ds(g * 16, 16)] = acc
        return carry2

      lax.fori_loop(0, ngrp, gbody, 0)
      pltpu.sync_copy(pv, pred.at[pl.ds(base, CHUNK)])
      return carry

    lax.fori_loop(0, nch, cbody, 0)

  return classifier


_DN = (((1,), (1,)), ((), ()))


def _enc_body(x_ref, w_ref, b_ref, o_ref):
  o_ref[...] = lax.dot_general(
      x_ref[...], w_ref[...], _DN,
      preferred_element_type=jnp.float32) + b_ref[...]


def _layer_body(x_ref, s_ref, cnt_ref, wl_ref, bl_ref, wr_ref, o_ref, *, relu):
  m = s_ref[...] / jnp.maximum(cnt_ref[...][:, 0:1], 1.0)
  y = lax.dot_general(x_ref[...], wl_ref[...], _DN,
                      preferred_element_type=jnp.float32)
  y = y + bl_ref[...]
  y = y + lax.dot_general(m, wr_ref[...], _DN,
                          preferred_element_type=jnp.float32)
  o_ref[...] = jnp.maximum(y, 0.0) if relu else y


def _tc_enc(x, w, b, R):
  npad, h = x.shape
  grid = npad // R
  return pl.pallas_call(
      _enc_body,
      grid=(grid,),
      in_specs=[
          pl.BlockSpec((R, h), lambda i: (i, 0)),
          pl.BlockSpec(w.shape, lambda i: (0, 0)),
          pl.BlockSpec((1, h), lambda i: (0, 0)),
      ],
      out_specs=pl.BlockSpec((R, h), lambda i: (i, 0)),
      out_shape=jax.ShapeDtypeStruct((npad, h), jnp.float32),
  )(x, w, b)


def _tc_layer(x, s, cnt, wl, bl, wr, R, relu):
  npad, h = x.shape
  grid = npad // R
  return pl.pallas_call(
      functools.partial(_layer_body, relu=relu),
      grid=(grid,),
      in_specs=[
          pl.BlockSpec((R, h), lambda i: (i, 0)),
          pl.BlockSpec((R, h), lambda i: (i, 0)),
          pl.BlockSpec((R, 16), lambda i: (i, 0)),
          pl.BlockSpec(wl.shape, lambda i: (0, 0)),
          pl.BlockSpec((1, h), lambda i: (0, 0)),
          pl.BlockSpec(wr.shape, lambda i: (0, 0)),
      ],
      out_specs=pl.BlockSpec((R, h), lambda i: (i, 0)),
      out_shape=jax.ShapeDtypeStruct((npad, h), jnp.float32),
  )(x, s, cnt, wl, bl, wr)


def kernel(x_doc, author_node_id, edge_index, edge_label_index,
           doc_W, doc_b, author_emb,
           c1_da_Wl, c1_da_bl, c1_da_Wr,
           c1_ad_Wl, c1_ad_bl, c1_ad_Wr,
           c2_da_Wl, c2_da_bl, c2_da_Wr,
           c2_ad_Wl, c2_ad_bl, c2_ad_Wr):
  ND, F = x_doc.shape
  H = doc_W.shape[0]
  NA = author_emb.shape[0]
  E = edge_index.shape[1]
  EL = edge_label_index.shape[1]

  NPAD = _round_up(max(ND, NA) + 1, NS * 64)
  PAD_ROW = max(ND, NA)
  EP_SEG = _round_up(E, NS * CHUNK)
  EP_CLS = _round_up(EL, NC * NS * CHUNK)
  GCH = 64
  R = NPAD // 5

  f32 = jnp.float32
  x_doc_p = jnp.pad(x_doc.astype(f32), ((0, NPAD - ND), (0, 0)))
  aid_p = jnp.pad(author_node_id.astype(jnp.int32), (0, NPAD - NA))
  src_p = jnp.pad(edge_index[0], (0, EP_SEG - E), constant_values=PAD_ROW)
  dst_p = jnp.pad(edge_index[1], (0, EP_SEG - E), constant_values=PAD_ROW)
  el0_p = jnp.pad(edge_label_index[0], (0, EP_CLS - EL))
  el1_p = jnp.pad(edge_label_index[1], (0, EP_CLS - EL))
  zrows = jnp.zeros((NPAD, H), f32)
  zcnt = jnp.zeros((NPAD, 16), f32)
  ones = jnp.ones((CHUNK, 16), f32)

  xd = _tc_enc(x_doc_p, doc_W, doc_b.reshape(1, H), R)
  xa = _make_emb_gather(NPAD, H, GCH)(author_emb, aid_p)

  seg1 = _make_seg_sum(NPAD, H, EP_SEG, with_counts=True)
  s_a1, s_d1, cnt_a, cnt_d = seg1(xd, xa, src_p, dst_p, zrows, zcnt, ones)

  xa1 = _tc_layer(xa, s_a1, cnt_a, c1_da_Wl, c1_da_bl.reshape(1, H),
                  c1_da_Wr, R, relu=True)
  xd1 = _tc_layer(xd, s_d1, cnt_d, c1_ad_Wl, c1_ad_bl.reshape(1, H),
                  c1_ad_Wr, R, relu=True)

  seg2 = _make_seg_sum(NPAD, H, EP_SEG, with_counts=False)
  s_a2, s_d2 = seg2(xd1, xa1, src_p, dst_p, zrows, zcnt, ones)

  xa2 = _tc_layer(xa1, s_a2, cnt_a, c2_da_Wl, c2_da_bl.reshape(1, H),
                  c2_da_Wr, R, relu=False)
  xd2 = _tc_layer(xd1, s_d2, cnt_d, c2_ad_Wl, c2_ad_bl.reshape(1, H),
                  c2_ad_Wr, R, relu=False)

  pred = _make_classifier(NPAD, H, EP_CLS)(xd2, xa2, el0_p, el1_p)
  return pred[:EL]


# speedup vs baseline: 1.8452x; 1.8452x over previous
import functools

import jax
import jax.numpy as jnp
from jax import lax
from jax.experimental import pallas as pl
from jax.experimental.pallas import tpu as pltpu
from jax.experimental.pallas import tpu_sc as plsc

NC = 2
NS = 16
CHUNK = 128

_mesh = functools.partial(
    plsc.VectorSubcoreMesh, core_axis_name="c", subcore_axis_name="s")
_params = functools.partial(pltpu.CompilerParams, needs_layout_passes=False)


def _round_up(x, m):
  return (x + m - 1) // m * m


def _make_emb_gather(NPAD, H, GCH):
  nch = NPAD // (NC * NS * GCH)

  @functools.partial(
      pl.kernel,
      out_type=jax.ShapeDtypeStruct((NPAD, H), jnp.float32),
      mesh=_mesh(),
      compiler_params=_params(),
      scratch_types=[
          pltpu.VMEM((GCH,), jnp.int32),
          pltpu.VMEM((GCH, H), jnp.float32),
          pltpu.SemaphoreType.DMA,
      ],
  )
  def emb_gather(table, idx, out, idx_v, rows_v, sem):
    wid = lax.axis_index("s") * NC + lax.axis_index("c")

    def body(i, carry):
      base = (wid * nch + i) * GCH
      pltpu.sync_copy(idx.at[pl.ds(base, GCH)], idx_v)
      pltpu.async_copy(table.at[idx_v], rows_v, sem).wait()
      pltpu.sync_copy(rows_v, out.at[pl.ds(base, GCH)])
      return carry

    lax.fori_loop(0, nch, body, 0)

  return emb_gather


def _make_counts(NPAD, H, EP):
  nch = EP // (NS * CHUNK)
  rpt = NPAD // NS

  @functools.partial(
      pl.kernel,
      out_type=[jax.ShapeDtypeStruct((NPAD, H), jnp.float32),
                jax.ShapeDtypeStruct((NPAD, H), jnp.float32)],
      mesh=_mesh(),
      compiler_params=_params(),
      scratch_types=[
          pltpu.VMEM_SHARED((NPAD, H), jnp.float32),
          pltpu.VMEM((CHUNK,), jnp.int32),
          pltpu.VMEM((CHUNK, H), jnp.float32),
      ],
  )
  def counts(dep, srcp, dstp, zrows, ones, cnt_a, cnt_b, cnt_sh, sidx,
             ones_v):
    del dep
    c = lax.axis_index("c")
    sid = lax.axis_index("s")
    r0 = sid * rpt

    pltpu.sync_copy(zrows, ones_v)
    for q in range(rpt // CHUNK):
      pltpu.sync_copy(ones_v, cnt_sh.at[pl.ds(r0 + q * CHUNK, CHUNK)])
    pltpu.sync_copy(ones, ones_v)
    plsc.subcore_barrier()

    def cbody(ch, carry):
      base = (sid * nch + ch) * CHUNK

      @pl.when(c == 0)
      def _():
        pltpu.sync_copy(dstp.at[pl.ds(base, CHUNK)], sidx)

      @pl.when(c == 1)
      def _():
        pltpu.sync_copy(srcp.at[pl.ds(base, CHUNK)], sidx)

      pltpu.sync_copy(ones_v, cnt_sh.at[sidx], add=True)
      return carry

    lax.fori_loop(0, nch, cbody, 0)
    plsc.subcore_barrier()

    @pl.when(c == 0)
    def _():
      for q in range(rpt // CHUNK):
        pltpu.sync_copy(cnt_sh.at[pl.ds(r0 + q * CHUNK, CHUNK)], ones_v)
        pltpu.sync_copy(ones_v, cnt_a.at[pl.ds(r0 + q * CHUNK, CHUNK)])

    @pl.when(c == 1)
    def _():
      for q in range(rpt // CHUNK):
        pltpu.sync_copy(cnt_sh.at[pl.ds(r0 + q * CHUNK, CHUNK)], ones_v)
        pltpu.sync_copy(ones_v, cnt_b.at[pl.ds(r0 + q * CHUNK, CHUNK)])

  return counts


def _make_seg_sum(NPAD, H, EP):
  nch = EP // (NS * CHUNK)
  rpt = NPAD // NS

  @functools.partial(
      pl.kernel,
      out_type=[jax.ShapeDtypeStruct((NPAD, H), jnp.float32),
                jax.ShapeDtypeStruct((NPAD, H), jnp.float32)],
      mesh=_mesh(),
      compiler_params=_params(),
      scratch_types=[
          pltpu.VMEM_SHARED((NPAD, H), jnp.float32),
          pltpu.VMEM((CHUNK,), jnp.int32),
          pltpu.VMEM((CHUNK,), jnp.int32),
          pltpu.VMEM((CHUNK, H), jnp.float32),
          pltpu.SemaphoreType.DMA,
      ],
  )
  def seg(dep, feat_a, feat_b, srcp, dstp, zrows, out_a, out_b,
          accum, gidx, sidx, rows, sem):
    del dep
    c = lax.axis_index("c")
    sid = lax.axis_index("s")
    r0 = sid * rpt

    pltpu.sync_copy(zrows, rows)
    for q in range(rpt // CHUNK):
      pltpu.sync_copy(rows, accum.at[pl.ds(r0 + q * CHUNK, CHUNK)])
    plsc.subcore_barrier()

    def cbody(ch, carry):
      base = (sid * nch + ch) * CHUNK

      @pl.when(c == 0)
      def _():
        pltpu.sync_copy(srcp.at[pl.ds(base, CHUNK)], gidx)
        pltpu.sync_copy(dstp.at[pl.ds(base, CHUNK)], sidx)
        pltpu.async_copy(feat_a.at[gidx], rows, sem).wait()

      @pl.when(c == 1)
      def _():
        pltpu.sync_copy(dstp.at[pl.ds(base, CHUNK)], gidx)
        pltpu.sync_copy(srcp.at[pl.ds(base, CHUNK)], sidx)
        pltpu.async_copy(feat_b.at[gidx], rows, sem).wait()

      pltpu.sync_copy(rows, accum.at[sidx], add=True)
      return carry

    lax.fori_loop(0, nch, cbody, 0)
    plsc.subcore_barrier()

    @pl.when(c == 0)
    def _():
      for q in range(rpt // CHUNK):
        pltpu.sync_copy(accum.at[pl.ds(r0 + q * CHUNK, CHUNK)], rows)
        pltpu.sync_copy(rows, out_a.at[pl.ds(r0 + q * CHUNK, CHUNK)])

    @pl.when(c == 1)
    def _():
      for q in range(rpt // CHUNK):
        pltpu.sync_copy(accum.at[pl.ds(r0 + q * CHUNK, CHUNK)], rows)
        pltpu.sync_copy(rows, out_b.at[pl.ds(r0 + q * CHUNK, CHUNK)])

  return seg


def _make_classifier(NPAD, H, EP):
  nch = EP // (NC * NS * CHUNK)
  ngrp = CHUNK // 16

  @functools.partial(
      pl.kernel,
      out_type=jax.ShapeDtypeStruct((EP,), jnp.float32),
      mesh=_mesh(),
      compiler_params=_params(),
      scratch_types=[
          pltpu.VMEM((CHUNK,), jnp.int32),
          pltpu.VMEM((CHUNK,), jnp.int32),
          pltpu.VMEM((CHUNK, H), jnp.float32),
          pltpu.VMEM((CHUNK, H), jnp.float32),
          pltpu.VMEM((CHUNK,), jnp.float32),
          pltpu.SemaphoreType.DMA,
          pltpu.SemaphoreType.DMA,
      ],
  )
  def classifier(xd, xa, el0, el1, pred, i0v, i1v, rd, ra, pv, s0, s1):
    wid = lax.axis_index("s") * NC + lax.axis_index("c")

    def cbody(ch, carry):
      base = (wid * nch + ch) * CHUNK
      pltpu.sync_copy(el0.at[pl.ds(base, CHUNK)], i0v)
      pltpu.sync_copy(el1.at[pl.ds(base, CHUNK)], i1v)
      cp0 = pltpu.async_copy(xd.at[i0v], rd, s0)
      cp1 = pltpu.async_copy(xa.at[i1v], ra, s1)
      cp0.wait()
      cp1.wait()

      def gbody(g, carry2):
        r16 = g * 16 + lax.iota(jnp.int32, 16)
        acc = jnp.zeros((16,), jnp.float32)
        for f in range(H):
          fv = jnp.full((16,), f, jnp.int32)
          acc = acc + (plsc.load_gather(rd, [r16, fv]) *
                       plsc.load_gather(ra, [r16, fv]))
        pv[pl.ds(g * 16, 16)] = acc
        return carry2

      lax.fori_loop(0, ngrp, gbody, 0)
      pltpu.sync_copy(pv, pred.at[pl.ds(base, CHUNK)])
      return carry

    lax.fori_loop(0, nch, cbody, 0)

  return classifier


_DN = (((1,), (1,)), ((), ()))


def _enc_body(x_ref, w_ref, b_ref, o_ref):
  o_ref[...] = lax.dot_general(
      x_ref[...], w_ref[...], _DN,
      preferred_element_type=jnp.float32) + b_ref[...]


def _layer_body(x_ref, s_ref, cnt_ref, wl_ref, bl_ref, wr_ref, o_ref, *, relu):
  m = s_ref[...] / jnp.maximum(cnt_ref[...][:, 0:1], 1.0)
  y = lax.dot_general(x_ref[...], wl_ref[...], _DN,
                      preferred_element_type=jnp.float32)
  y = y + bl_ref[...]
  y = y + lax.dot_general(m, wr_ref[...], _DN,
                          preferred_element_type=jnp.float32)
  o_ref[...] = jnp.maximum(y, 0.0) if relu else y


def _tc_enc(x, w, b, R):
  npad, h = x.shape
  grid = npad // R
  return pl.pallas_call(
      _enc_body,
      grid=(grid,),
      in_specs=[
          pl.BlockSpec((R, h), lambda i: (i, 0)),
          pl.BlockSpec(w.shape, lambda i: (0, 0)),
          pl.BlockSpec((1, h), lambda i: (0, 0)),
      ],
      out_specs=pl.BlockSpec((R, h), lambda i: (i, 0)),
      out_shape=jax.ShapeDtypeStruct((npad, h), jnp.float32),
  )(x, w, b)


def _tc_layer(x, s, cnt, wl, bl, wr, R, relu):
  npad, h = x.shape
  grid = npad // R
  return pl.pallas_call(
      functools.partial(_layer_body, relu=relu),
      grid=(grid,),
      in_specs=[
          pl.BlockSpec((R, h), lambda i: (i, 0)),
          pl.BlockSpec((R, h), lambda i: (i, 0)),
          pl.BlockSpec((R, h), lambda i: (i, 0)),
          pl.BlockSpec(wl.shape, lambda i: (0, 0)),
          pl.BlockSpec((1, h), lambda i: (0, 0)),
          pl.BlockSpec(wr.shape, lambda i: (0, 0)),
      ],
      out_specs=pl.BlockSpec((R, h), lambda i: (i, 0)),
      out_shape=jax.ShapeDtypeStruct((npad, h), jnp.float32),
  )(x, s, cnt, wl, bl, wr)


def kernel(x_doc, author_node_id, edge_index, edge_label_index,
           doc_W, doc_b, author_emb,
           c1_da_Wl, c1_da_bl, c1_da_Wr,
           c1_ad_Wl, c1_ad_bl, c1_ad_Wr,
           c2_da_Wl, c2_da_bl, c2_da_Wr,
           c2_ad_Wl, c2_ad_bl, c2_ad_Wr):
  ND, F = x_doc.shape
  H = doc_W.shape[0]
  NA = author_emb.shape[0]
  E = edge_index.shape[1]
  EL = edge_label_index.shape[1]

  NPAD = _round_up(max(ND, NA) + 1, NS * CHUNK)
  PAD_ROW = max(ND, NA)
  EP_SEG = _round_up(E, NS * CHUNK)
  EP_CLS = _round_up(EL, NC * NS * CHUNK)
  GCH = 64
  R = NPAD // 5

  f32 = jnp.float32
  x_doc_p = jnp.pad(x_doc.astype(f32), ((0, NPAD - ND), (0, 0)))
  aid_p = jnp.pad(author_node_id.astype(jnp.int32), (0, NPAD - NA))
  src_p = jnp.pad(edge_index[0], (0, EP_SEG - E), constant_values=PAD_ROW)
  dst_p = jnp.pad(edge_index[1], (0, EP_SEG - E), constant_values=PAD_ROW)
  el0_p = jnp.pad(edge_label_index[0], (0, EP_CLS - EL))
  el1_p = jnp.pad(edge_label_index[1], (0, EP_CLS - EL))
  zrows = jnp.zeros((CHUNK, H), f32)
  ones = jnp.ones((CHUNK, H), f32)

  xd = _tc_enc(x_doc_p, doc_W, doc_b.reshape(1, H), R)
  xa = _make_emb_gather(NPAD, H, GCH)(author_emb, aid_p)
  cnt_a, cnt_d = _make_counts(NPAD, H, EP_SEG)(xa, src_p, dst_p, zrows, ones)

  seg = _make_seg_sum(NPAD, H, EP_SEG)
  s_a1, s_d1 = seg(cnt_a, xd, xa, src_p, dst_p, zrows)

  xa1 = _tc_layer(xa, s_a1, cnt_a, c1_da_Wl, c1_da_bl.reshape(1, H),
                  c1_da_Wr, R, relu=True)
  xd1 = _tc_layer(xd, s_d1, cnt_d, c1_ad_Wl, c1_ad_bl.reshape(1, H),
                  c1_ad_Wr, R, relu=True)

  s_a2, s_d2 = seg(s_a1, xd1, xa1, src_p, dst_p, zrows)

  xa2 = _tc_layer(xa1, s_a2, cnt_a, c2_da_Wl, c2_da_bl.reshape(1, H),
                  c2_da_Wr, R, relu=False)
  xd2 = _tc_layer(xd1, s_d2, cnt_d, c2_ad_Wl, c2_ad_bl.reshape(1, H),
                  c2_ad_Wr, R, relu=False)

  pred = _make_classifier(NPAD, H, EP_CLS)(xd2, xa2, el0_p, el1_p)
  return pred[:EL]
